# baseline (speedup 1.0000x reference)
import functools
import math

import jax
import jax.numpy as jnp
import numpy as np
from jax.experimental import pallas as pl
from jax.experimental.pallas import tpu as pltpu

HIDDEN = 128
NHEADS = 16
DH = HIDDEN // NHEADS
NRG = 16
TI = 64
TJ = 64

_OFF = np.linspace(0.0, 10.0, NRG).astype(np.float32)
_COEFF = np.float32(-0.5 / (_OFF[1] - _OFF[0]) ** 2)


def _ln(z, g, be):
    mu = z.mean(-1, keepdims=True)
    var = jnp.mean(z * z, axis=-1, keepdims=True) - mu * mu
    return (z - mu) * (1.0 / jnp.sqrt(var + 1e-5)) * g + be


def _pick_tile(n, cap=1024):
    best = 8
    for d in range(8, cap + 1, 8):
        if n % d == 0:
            best = d
    return best


def _pre_kernel(h_ref, w_ref, vec_ref, q_ref, ahk_ref, ahv_ref, axv_ref,
                bhk_ref, bhv_ref, bxv_ref):
    h = h_ref[...]
    w = w_ref
    b1q = vec_ref[0:1, :]
    gq = vec_ref[1:2, :]
    beq = vec_ref[2:3, :]
    b2q = vec_ref[3:4, :]
    dot = functools.partial(jnp.dot, preferred_element_type=jnp.float32)
    z = dot(h, w[0:128, :]) + b1q
    z = jax.nn.relu(_ln(z, gq, beq))
    q_ref[...] = (dot(z, w[128:256, :]) + b2q) * (1.0 / np.sqrt(DH))
    ahk_ref[...] = dot(h, w[256:384, :])
    ahv_ref[...] = dot(h, w[384:512, :])
    axv_ref[...] = dot(h, w[512:640, :])
    bhk_ref[...] = dot(h, w[640:768, :]) + vec_ref[4:5, :]
    bhv_ref[...] = dot(h, w[768:896, :]) + vec_ref[5:6, :]
    bxv_ref[...] = dot(h, w[896:1024, :]) + vec_ref[6:7, :]


def _pair_kernel(start_ref, num_ref,
                 q_ref, bhk_ref, bhv_ref, bxv_ref, hdst_ref, xdst_ref, auxd_ref,
                 ahk_ref, ahv_ref, axv_ref, xsrc_ref, auxs_ref,
                 w1r_ref, w2hk_ref, w2hv_ref, w2xv_ref,
                 wno1a_ref, wno1b_ref, wno2_ref, vec_ref, h01_ref,
                 hout_ref, xout_ref):
    t = pl.program_id(0)
    start = start_ref[t]
    nst = num_ref[t]

    dot = functools.partial(jnp.dot, preferred_element_type=jnp.float32)

    q = q_ref[...]
    bhk_i = bhk_ref[...]
    bhv_i = bhv_ref[...]
    bxv_i = bxv_ref[...]
    auxd = auxd_ref[...]
    b_i = auxd[:, 0:1]
    ml_i = auxd[:, 1:2]
    bi_eff = jnp.where(ml_i == 1, b_i, -1)
    bi_rep = jnp.broadcast_to(bi_eff, (TI, HIDDEN))
    xd = xdst_ref[...]
    xi0 = xd[:, 0:1]
    xi1 = xd[:, 1:2]
    xi2 = xd[:, 2:3]
    xi0_rep = jnp.broadcast_to(xi0, (TI, HIDDEN))
    xi1_rep = jnp.broadcast_to(xi1, (TI, HIDDEN))
    xi2_rep = jnp.broadcast_to(xi2, (TI, HIDDEN))

    g_hk = vec_ref[0:1, :]
    be_hk = vec_ref[1:2, :]
    b2_hk = vec_ref[2:3, :]
    g_hv = vec_ref[3:4, :]
    be_hv = vec_ref[4:5, :]
    b2_hv = vec_ref[5:6, :]
    g_xv = vec_ref[6:7, :]
    be_xv = vec_ref[7:8, :]
    b2_xv_rep = vec_ref[8:9, :]
    b1_no = vec_ref[9:10, :]
    g_no = vec_ref[10:11, :]
    be_no = vec_ref[11:12, :]
    b2_no = vec_ref[12:13, :]

    lane = jax.lax.broadcasted_iota(jnp.int32, (1, HIDDEN), 1)
    off_rep = (lane // DH).astype(jnp.float32) * np.float32(10.0 / (NRG - 1))
    rr = jax.lax.broadcasted_iota(jnp.int32, (HIDDEN, HIDDEN), 0) // DH
    cc = jax.lax.broadcasted_iota(jnp.int32, (HIDDEN, HIDDEN), 1) // DH
    s_head = (rr == cc).astype(jnp.float32)

    def body(s, carry):
        j0 = (start + s) * TJ
        auxj = auxs_ref[pl.ds(j0, TJ), :]
        bj_eff = jnp.where(auxj[:, 1:2] == 0, auxj[:, 0:1], -2)
        bj_rep = jnp.broadcast_to(bj_eff, (TJ, HIDDEN))
        mask3 = bi_rep[:, None, :] == bj_rep[None, :, :]
        return jax.lax.cond(
            jnp.any(mask3), lambda c: _tile(j0, mask3, c), lambda c: c, carry)

    def _tile(j0, mask3, carry):
        m, l, accv, ax0, ax1, ax2 = carry
        ahk = ahk_ref[pl.ds(j0, TJ), :]
        ahv = ahv_ref[pl.ds(j0, TJ), :]
        axv = axv_ref[pl.ds(j0, TJ), :]
        xj = xsrc_ref[pl.ds(j0, TJ), :]
        xj0_rep = jnp.broadcast_to(xj[:, 0:1], (TJ, HIDDEN))
        xj1_rep = jnp.broadcast_to(xj[:, 1:2], (TJ, HIDDEN))
        xj2_rep = jnp.broadcast_to(xj[:, 2:3], (TJ, HIDDEN))

        pen3 = jnp.where(mask3, 0.0, -jnp.inf)
        rel0 = xi0_rep[:, None, :] - xj0_rep[None, :, :]
        rel1 = xi1_rep[:, None, :] - xj1_rep[None, :, :]
        rel2 = xi2_rep[:, None, :] - xj2_rep[None, :, :]
        dist = jnp.sqrt(rel0 * rel0 + rel1 * rel1 + rel2 * rel2)
        rf = jnp.exp(_COEFF * (dist - off_rep[None, :, :]) ** 2)
        r_all = dot(rf.reshape(TI * TJ, HIDDEN), w1r_ref[...])
        r_all = r_all.reshape(TI, TJ, 3 * HIDDEN)

        z = r_all[:, :, 0:HIDDEN] + ahk[None, :, :] + bhk_i[:, None, :]
        z = jax.nn.relu(_ln(z, g_hk, be_hk))
        k = dot(z.reshape(TI * TJ, HIDDEN), w2hk_ref[...]) + b2_hk
        qk = (q[:, None, :] * k.reshape(TI, TJ, HIDDEN)).reshape(TI * TJ, HIDDEN)
        logit = dot(qk, s_head).reshape(TI, TJ, HIDDEN) + pen3

        tmax = logit.max(axis=1)
        m_new = jnp.maximum(m, tmax)
        m_sub = jnp.where(jnp.isfinite(m_new), m_new, 0.0)
        m_sub_prev = jnp.where(jnp.isfinite(m), m, 0.0)
        scale = jnp.where(jnp.isfinite(m), jnp.exp(m_sub_prev - m_sub), 0.0)
        p = jnp.exp(logit - m_sub[:, None, :])
        l_new = l * scale + p.sum(axis=1)

        z = r_all[:, :, HIDDEN:2 * HIDDEN] + ahv[None, :, :] + bhv_i[:, None, :]
        z = jax.nn.relu(_ln(z, g_hv, be_hv))
        v = dot(z.reshape(TI * TJ, HIDDEN), w2hv_ref[...]) + b2_hv
        pv = (p * v.reshape(TI, TJ, HIDDEN)).sum(axis=1)
        accv_new = accv * scale + pv

        z = r_all[:, :, 2 * HIDDEN:3 * HIDDEN] + axv[None, :, :] + bxv_i[:, None, :]
        z = jax.nn.relu(_ln(z, g_xv, be_xv))
        xv = dot(z.reshape(TI * TJ, HIDDEN), w2xv_ref[...]) + b2_xv_rep
        w = p * xv.reshape(TI, TJ, HIDDEN)
        ax0_new = ax0 * scale + (w * rel0).sum(axis=1)
        ax1_new = ax1 * scale + (w * rel1).sum(axis=1)
        ax2_new = ax2 * scale + (w * rel2).sum(axis=1)
        return m_new, l_new, accv_new, ax0_new, ax1_new, ax2_new

    init = (jnp.full((TI, HIDDEN), -jnp.inf, jnp.float32),
            jnp.zeros((TI, HIDDEN), jnp.float32),
            jnp.zeros((TI, HIDDEN), jnp.float32),
            jnp.zeros((TI, HIDDEN), jnp.float32),
            jnp.zeros((TI, HIDDEN), jnp.float32),
            jnp.zeros((TI, HIDDEN), jnp.float32))
    m, l, accv, ax0, ax1, ax2 = jax.lax.fori_loop(0, nst, body, init)

    denom = l + 1e-16
    attn_out = accv / denom
    dx0 = (ax0 / denom).sum(axis=1, keepdims=True) * np.float32(1.0 / HIDDEN)
    dx1 = (ax1 / denom).sum(axis=1, keepdims=True) * np.float32(1.0 / HIDDEN)
    dx2 = (ax2 / denom).sum(axis=1, keepdims=True) * np.float32(1.0 / HIDDEN)
    xout_ref[...] = jnp.concatenate(
        [xi0 + dx0, xi1 + dx1, xi2 + dx2, xd[:, 3:8]], axis=1)

    hml = jnp.where(ml_i == 1, h01_ref[1:2, :], h01_ref[0:1, :])
    z = (dot(attn_out, wno1a_ref[...]) + dot(hml, wno1b_ref[...]) + b1_no)
    z = jax.nn.relu(_ln(z, g_no, be_no))
    hout_ref[...] = dot(z, wno2_ref[...]) + b2_no + hdst_ref[...]


def kernel(h, x, params, batch, edge_index, mask_ligand):
    n = h.shape[0]
    npad = -(-n // TJ) * TJ
    num_t = npad // TI
    pad = npad - n

    f32 = jnp.float32
    h = h.astype(f32)
    x = x.astype(f32)
    b32 = batch.astype(jnp.int32)
    ml32 = mask_ligand.astype(jnp.int32)

    perm = jnp.argsort(b32 * 2 + ml32, stable=True)
    inv = jnp.zeros((n,), jnp.int32).at[perm].set(
        jnp.arange(n, dtype=jnp.int32))
    hs = jnp.take(h, perm, axis=0)
    xs = jnp.take(x, perm, axis=0)
    bs = jnp.take(b32, perm)
    mls = jnp.take(ml32, perm)

    sentinel = np.int32(1 << 20)
    hp = jnp.pad(hs, ((0, pad), (0, 0)))
    xp = jnp.pad(xs, ((0, pad), (0, 5)))
    batch_p = jnp.pad(bs, (0, pad), constant_values=sentinel)
    ml_p = jnp.pad(mls, (0, pad))
    zcol = jnp.zeros((npad, 6), jnp.int32)
    auxd = jnp.concatenate([batch_p[:, None], ml_p[:, None], zcol], axis=1)

    p = params
    wpack = jnp.concatenate([
        p['hq_W1'], p['hq_W2'],
        p['hk_W1'][NRG:NRG + HIDDEN], p['hv_W1'][NRG:NRG + HIDDEN],
        p['xv_W1'][NRG:NRG + HIDDEN],
        p['hk_W1'][NRG + HIDDEN:], p['hv_W1'][NRG + HIDDEN:],
        p['xv_W1'][NRG + HIDDEN:],
    ], axis=0)
    vec_pre = jnp.stack([
        p['hq_b1'], p['hq_g'], p['hq_be'], p['hq_b2'],
        p['hk_b1'], p['hv_b1'], p['xv_b1'], jnp.zeros((HIDDEN,), f32)], axis=0)

    tp = _pick_tile(npad)
    node_out = tuple(jax.ShapeDtypeStruct((npad, HIDDEN), f32) for _ in range(7))
    blk = pl.BlockSpec((tp, HIDDEN), lambda i: (i, 0))
    q, ahk, ahv, axv, bhk, bhv, bxv = pl.pallas_call(
        _pre_kernel,
        grid=(npad // tp,),
        in_specs=[blk,
                  pl.BlockSpec((8 * HIDDEN, HIDDEN), lambda i: (0, 0)),
                  pl.BlockSpec((8, HIDDEN), lambda i: (0, 0))],
        out_specs=tuple(blk for _ in range(7)),
        out_shape=node_out,
    )(hp, wpack, vec_pre)

    key_p = batch_p * 2 + ml_p
    i0 = jnp.arange(num_t, dtype=jnp.int32) * TI
    g_lo = batch_p[i0]
    g_hi = batch_p[i0 + TI - 1]
    row_lo = jnp.searchsorted(key_p, 2 * g_lo, side='left').astype(jnp.int32)
    row_hi = jnp.searchsorted(key_p, 2 * g_hi + 1, side='left').astype(jnp.int32)
    start_t = row_lo // TJ
    num_s = -(-row_hi // TJ) - start_t
    has_lig = ml_p.reshape(num_t, TI).max(axis=1) > 0
    num_s = jnp.where(has_lig, num_s, 0)

    w1r = jnp.concatenate(
        [p['hk_W1'][:NRG], p['hv_W1'][:NRG], p['xv_W1'][:NRG]], axis=1)
    w1r_rep = jnp.broadcast_to(
        w1r[:, None, :] * np.float32(1.0 / DH),
        (NRG, DH, 3 * HIDDEN)).reshape(HIDDEN, 3 * HIDDEN)
    w2xv_rep = jnp.broadcast_to(
        p['xv_W2'][:, :, None], (HIDDEN, NHEADS, DH)).reshape(HIDDEN, HIDDEN)
    b2xv_rep = jnp.broadcast_to(
        p['xv_b2'][:, None], (NHEADS, DH)).reshape(HIDDEN)
    vec_pair = jnp.stack([
        p['hk_g'], p['hk_be'], p['hk_b2'],
        p['hv_g'], p['hv_be'], p['hv_b2'],
        p['xv_g'], p['xv_be'], b2xv_rep,
        p['no_b1'], p['no_g'], p['no_be'], p['no_b2'],
        jnp.zeros((HIDDEN,), f32), jnp.zeros((HIDDEN,), f32),
        jnp.zeros((HIDDEN,), f32)], axis=0)
    h01 = jnp.pad(h[0:2], ((0, 6), (0, 0)))

    dstH = pl.BlockSpec((TI, HIDDEN), lambda t, s0, s1: (t, 0))
    dst8 = pl.BlockSpec((TI, 8), lambda t, s0, s1: (t, 0))
    full = lambda r, c: pl.BlockSpec((r, c), lambda t, s0, s1: (0, 0))

    grid_spec = pltpu.PrefetchScalarGridSpec(
        num_scalar_prefetch=2,
        grid=(num_t,),
        in_specs=[dstH, dstH, dstH, dstH, dstH, dst8, dst8,
                  full(npad, HIDDEN), full(npad, HIDDEN), full(npad, HIDDEN),
                  full(npad, 8), full(npad, 8),
                  full(HIDDEN, 3 * HIDDEN), full(HIDDEN, HIDDEN),
                  full(HIDDEN, HIDDEN), full(HIDDEN, HIDDEN),
                  full(HIDDEN, HIDDEN), full(HIDDEN, HIDDEN),
                  full(HIDDEN, HIDDEN), full(16, HIDDEN), full(8, HIDDEN)],
        out_specs=[dstH, dst8],
    )
    hout, xout = pl.pallas_call(
        _pair_kernel,
        grid_spec=grid_spec,
        out_shape=(jax.ShapeDtypeStruct((npad, HIDDEN), f32),
                   jax.ShapeDtypeStruct((npad, 8), f32)),
        compiler_params=pltpu.CompilerParams(
            dimension_semantics=("arbitrary",),
            vmem_limit_bytes=128 * 1024 * 1024),
    )(start_t, num_s,
      q, bhk, bhv, bxv, hp, xp, auxd,
      ahk, ahv, axv, xp, auxd,
      w1r_rep, p['hk_W2'], p['hv_W2'], w2xv_rep,
      p['no_W1'][:HIDDEN], p['no_W1'][HIDDEN:], p['no_W2'], vec_pair, h01)

    return jnp.take(hout[:n], inv, axis=0), jnp.take(xout[:n, :3], inv, axis=0)


# speedup vs baseline: 1.1524x; 1.1524x over previous
import functools
import math

import jax
import jax.numpy as jnp
import numpy as np
from jax.experimental import pallas as pl
from jax.experimental.pallas import tpu as pltpu

HIDDEN = 128
NHEADS = 16
DH = HIDDEN // NHEADS
NRG = 16
TI = 32
TJ = 64

_OFF = np.linspace(0.0, 10.0, NRG).astype(np.float32)
_COEFF = np.float32(-0.5 / (_OFF[1] - _OFF[0]) ** 2)


def _ln(z, g, be):
    mu = z.mean(-1, keepdims=True)
    var = jnp.mean(z * z, axis=-1, keepdims=True) - mu * mu
    return (z - mu) * (1.0 / jnp.sqrt(var + 1e-5)) * g + be


def _pick_tile(n, cap=1024):
    best = 8
    for d in range(8, cap + 1, 8):
        if n % d == 0:
            best = d
    return best


def _pre_kernel(h_ref, w_ref, vec_ref, q_ref, ahk_ref, ahv_ref, axv_ref,
                bhk_ref, bhv_ref, bxv_ref):
    h = h_ref[...]
    w = w_ref
    b1q = vec_ref[0:1, :]
    gq = vec_ref[1:2, :]
    beq = vec_ref[2:3, :]
    b2q = vec_ref[3:4, :]
    dot = functools.partial(jnp.dot, preferred_element_type=jnp.float32)
    z = dot(h, w[0:128, :]) + b1q
    z = jax.nn.relu(_ln(z, gq, beq))
    q_ref[...] = (dot(z, w[128:256, :]) + b2q) * (1.0 / np.sqrt(DH))
    ahk_ref[...] = dot(h, w[256:384, :])
    ahv_ref[...] = dot(h, w[384:512, :])
    axv_ref[...] = dot(h, w[512:640, :])
    bhk_ref[...] = dot(h, w[640:768, :]) + vec_ref[4:5, :]
    bhv_ref[...] = dot(h, w[768:896, :]) + vec_ref[5:6, :]
    bxv_ref[...] = dot(h, w[896:1024, :]) + vec_ref[6:7, :]


def _pair_kernel(start_ref, num_ref,
                 q_ref, bhk_ref, bhv_ref, bxv_ref, hdst_ref, xdst_ref, auxd_ref,
                 ahk_ref, ahv_ref, axv_ref, xsrc_ref, auxs_ref,
                 w1r_ref, w2hk_ref, w2hv_ref, w2xv_ref,
                 wno1a_ref, wno1b_ref, wno2_ref, vec_ref, h01_ref,
                 hout_ref, xout_ref):
    t = pl.program_id(0)
    start = start_ref[t]
    nst = num_ref[t]

    dot = functools.partial(jnp.dot, preferred_element_type=jnp.float32)

    q = q_ref[...]
    bhk_i = bhk_ref[...]
    bhv_i = bhv_ref[...]
    bxv_i = bxv_ref[...]
    auxd = auxd_ref[...]
    b_i = auxd[:, 0:1]
    ml_i = auxd[:, 1:2]
    bi_eff = jnp.where(ml_i == 1, b_i, -1)
    bi_rep = jnp.broadcast_to(bi_eff, (TI, HIDDEN))
    xd = xdst_ref[...]
    xi0 = xd[:, 0:1]
    xi1 = xd[:, 1:2]
    xi2 = xd[:, 2:3]
    xi0_rep = jnp.broadcast_to(xi0, (TI, HIDDEN))
    xi1_rep = jnp.broadcast_to(xi1, (TI, HIDDEN))
    xi2_rep = jnp.broadcast_to(xi2, (TI, HIDDEN))

    g_hk = vec_ref[0:1, :]
    be_hk = vec_ref[1:2, :]
    b2_hk = vec_ref[2:3, :]
    g_hv = vec_ref[3:4, :]
    be_hv = vec_ref[4:5, :]
    b2_hv = vec_ref[5:6, :]
    g_xv = vec_ref[6:7, :]
    be_xv = vec_ref[7:8, :]
    b2_xv_rep = vec_ref[8:9, :]
    b1_no = vec_ref[9:10, :]
    g_no = vec_ref[10:11, :]
    be_no = vec_ref[11:12, :]
    b2_no = vec_ref[12:13, :]

    lane = jax.lax.broadcasted_iota(jnp.int32, (1, HIDDEN), 1)
    off_rep = (lane // DH).astype(jnp.float32) * np.float32(10.0 / (NRG - 1))
    rr = jax.lax.broadcasted_iota(jnp.int32, (HIDDEN, HIDDEN), 0) // DH
    cc = jax.lax.broadcasted_iota(jnp.int32, (HIDDEN, HIDDEN), 1) // DH
    s_head = (rr == cc).astype(jnp.float32)

    def body(s, carry):
        j0 = (start + s) * TJ
        auxj = auxs_ref[pl.ds(j0, TJ), :]
        bj_eff = jnp.where(auxj[:, 1:2] == 0, auxj[:, 0:1], -2)
        bj_rep = jnp.broadcast_to(bj_eff, (TJ, HIDDEN))
        mask3 = bi_rep[:, None, :] == bj_rep[None, :, :]
        return jax.lax.cond(
            jnp.any(mask3), lambda c: _tile(j0, mask3, c), lambda c: c, carry)

    def _tile(j0, mask3, carry):
        m, l, accv, ax0, ax1, ax2 = carry
        ahk = ahk_ref[pl.ds(j0, TJ), :]
        ahv = ahv_ref[pl.ds(j0, TJ), :]
        axv = axv_ref[pl.ds(j0, TJ), :]
        xj = xsrc_ref[pl.ds(j0, TJ), :]
        xj0_rep = jnp.broadcast_to(xj[:, 0:1], (TJ, HIDDEN))
        xj1_rep = jnp.broadcast_to(xj[:, 1:2], (TJ, HIDDEN))
        xj2_rep = jnp.broadcast_to(xj[:, 2:3], (TJ, HIDDEN))

        pen3 = jnp.where(mask3, 0.0, -jnp.inf)
        rel0 = xi0_rep[:, None, :] - xj0_rep[None, :, :]
        rel1 = xi1_rep[:, None, :] - xj1_rep[None, :, :]
        rel2 = xi2_rep[:, None, :] - xj2_rep[None, :, :]
        dist = jnp.sqrt(rel0 * rel0 + rel1 * rel1 + rel2 * rel2)
        rf = jnp.exp(_COEFF * (dist - off_rep[None, :, :]) ** 2)
        r_all = dot(rf.reshape(TI * TJ, HIDDEN), w1r_ref[...])
        r_all = r_all.reshape(TI, TJ, 3 * HIDDEN)

        z = r_all[:, :, 0:HIDDEN] + ahk[None, :, :] + bhk_i[:, None, :]
        z = jax.nn.relu(_ln(z, g_hk, be_hk))
        k = dot(z.reshape(TI * TJ, HIDDEN), w2hk_ref[...]) + b2_hk
        qk = (q[:, None, :] * k.reshape(TI, TJ, HIDDEN)).reshape(TI * TJ, HIDDEN)
        logit = dot(qk, s_head).reshape(TI, TJ, HIDDEN) + pen3

        tmax = logit.max(axis=1)
        m_new = jnp.maximum(m, tmax)
        m_sub = jnp.where(jnp.isfinite(m_new), m_new, 0.0)
        m_sub_prev = jnp.where(jnp.isfinite(m), m, 0.0)
        scale = jnp.where(jnp.isfinite(m), jnp.exp(m_sub_prev - m_sub), 0.0)
        p = jnp.exp(logit - m_sub[:, None, :])
        l_new = l * scale + p.sum(axis=1)

        z = r_all[:, :, HIDDEN:2 * HIDDEN] + ahv[None, :, :] + bhv_i[:, None, :]
        z = jax.nn.relu(_ln(z, g_hv, be_hv))
        v = dot(z.reshape(TI * TJ, HIDDEN), w2hv_ref[...]) + b2_hv
        pv = (p * v.reshape(TI, TJ, HIDDEN)).sum(axis=1)
        accv_new = accv * scale + pv

        z = r_all[:, :, 2 * HIDDEN:3 * HIDDEN] + axv[None, :, :] + bxv_i[:, None, :]
        z = jax.nn.relu(_ln(z, g_xv, be_xv))
        xv = dot(z.reshape(TI * TJ, HIDDEN), w2xv_ref[...]) + b2_xv_rep
        w = p * xv.reshape(TI, TJ, HIDDEN)
        ax0_new = ax0 * scale + (w * rel0).sum(axis=1)
        ax1_new = ax1 * scale + (w * rel1).sum(axis=1)
        ax2_new = ax2 * scale + (w * rel2).sum(axis=1)
        return m_new, l_new, accv_new, ax0_new, ax1_new, ax2_new

    init = (jnp.full((TI, HIDDEN), -jnp.inf, jnp.float32),
            jnp.zeros((TI, HIDDEN), jnp.float32),
            jnp.zeros((TI, HIDDEN), jnp.float32),
            jnp.zeros((TI, HIDDEN), jnp.float32),
            jnp.zeros((TI, HIDDEN), jnp.float32),
            jnp.zeros((TI, HIDDEN), jnp.float32))
    m, l, accv, ax0, ax1, ax2 = jax.lax.fori_loop(0, nst, body, init)

    denom = l + 1e-16
    attn_out = accv / denom
    dx0 = (ax0 / denom).sum(axis=1, keepdims=True) * np.float32(1.0 / HIDDEN)
    dx1 = (ax1 / denom).sum(axis=1, keepdims=True) * np.float32(1.0 / HIDDEN)
    dx2 = (ax2 / denom).sum(axis=1, keepdims=True) * np.float32(1.0 / HIDDEN)
    xout_ref[...] = jnp.concatenate(
        [xi0 + dx0, xi1 + dx1, xi2 + dx2, xd[:, 3:8]], axis=1)

    hml = jnp.where(ml_i == 1, h01_ref[1:2, :], h01_ref[0:1, :])
    z = (dot(attn_out, wno1a_ref[...]) + dot(hml, wno1b_ref[...]) + b1_no)
    z = jax.nn.relu(_ln(z, g_no, be_no))
    hout_ref[...] = dot(z, wno2_ref[...]) + b2_no + hdst_ref[...]


def kernel(h, x, params, batch, edge_index, mask_ligand):
    n = h.shape[0]
    npad = -(-n // TJ) * TJ
    num_t = npad // TI
    pad = npad - n

    f32 = jnp.float32
    h = h.astype(f32)
    x = x.astype(f32)
    b32 = batch.astype(jnp.int32)
    ml32 = mask_ligand.astype(jnp.int32)

    perm = jnp.argsort(b32 * 2 + ml32, stable=True)
    inv = jnp.zeros((n,), jnp.int32).at[perm].set(
        jnp.arange(n, dtype=jnp.int32))
    hs = jnp.take(h, perm, axis=0)
    xs = jnp.take(x, perm, axis=0)
    bs = jnp.take(b32, perm)
    mls = jnp.take(ml32, perm)

    sentinel = np.int32(1 << 20)
    hp = jnp.pad(hs, ((0, pad), (0, 0)))
    xp = jnp.pad(xs, ((0, pad), (0, 5)))
    batch_p = jnp.pad(bs, (0, pad), constant_values=sentinel)
    ml_p = jnp.pad(mls, (0, pad))
    zcol = jnp.zeros((npad, 6), jnp.int32)
    auxd = jnp.concatenate([batch_p[:, None], ml_p[:, None], zcol], axis=1)

    p = params
    wpack = jnp.concatenate([
        p['hq_W1'], p['hq_W2'],
        p['hk_W1'][NRG:NRG + HIDDEN], p['hv_W1'][NRG:NRG + HIDDEN],
        p['xv_W1'][NRG:NRG + HIDDEN],
        p['hk_W1'][NRG + HIDDEN:], p['hv_W1'][NRG + HIDDEN:],
        p['xv_W1'][NRG + HIDDEN:],
    ], axis=0)
    vec_pre = jnp.stack([
        p['hq_b1'], p['hq_g'], p['hq_be'], p['hq_b2'],
        p['hk_b1'], p['hv_b1'], p['xv_b1'], jnp.zeros((HIDDEN,), f32)], axis=0)

    tp = _pick_tile(npad)
    node_out = tuple(jax.ShapeDtypeStruct((npad, HIDDEN), f32) for _ in range(7))
    blk = pl.BlockSpec((tp, HIDDEN), lambda i: (i, 0))
    q, ahk, ahv, axv, bhk, bhv, bxv = pl.pallas_call(
        _pre_kernel,
        grid=(npad // tp,),
        in_specs=[blk,
                  pl.BlockSpec((8 * HIDDEN, HIDDEN), lambda i: (0, 0)),
                  pl.BlockSpec((8, HIDDEN), lambda i: (0, 0))],
        out_specs=tuple(blk for _ in range(7)),
        out_shape=node_out,
    )(hp, wpack, vec_pre)

    key_p = batch_p * 2 + ml_p
    i0 = jnp.arange(num_t, dtype=jnp.int32) * TI
    g_lo = batch_p[i0]
    g_hi = batch_p[i0 + TI - 1]
    row_lo = jnp.searchsorted(key_p, 2 * g_lo, side='left').astype(jnp.int32)
    row_hi = jnp.searchsorted(key_p, 2 * g_hi + 1, side='left').astype(jnp.int32)
    start_t = row_lo // TJ
    num_s = -(-row_hi // TJ) - start_t
    has_lig = ml_p.reshape(num_t, TI).max(axis=1) > 0
    num_s = jnp.where(has_lig, num_s, 0)

    w1r = jnp.concatenate(
        [p['hk_W1'][:NRG], p['hv_W1'][:NRG], p['xv_W1'][:NRG]], axis=1)
    w1r_rep = jnp.broadcast_to(
        w1r[:, None, :] * np.float32(1.0 / DH),
        (NRG, DH, 3 * HIDDEN)).reshape(HIDDEN, 3 * HIDDEN)
    w2xv_rep = jnp.broadcast_to(
        p['xv_W2'][:, :, None], (HIDDEN, NHEADS, DH)).reshape(HIDDEN, HIDDEN)
    b2xv_rep = jnp.broadcast_to(
        p['xv_b2'][:, None], (NHEADS, DH)).reshape(HIDDEN)
    vec_pair = jnp.stack([
        p['hk_g'], p['hk_be'], p['hk_b2'],
        p['hv_g'], p['hv_be'], p['hv_b2'],
        p['xv_g'], p['xv_be'], b2xv_rep,
        p['no_b1'], p['no_g'], p['no_be'], p['no_b2'],
        jnp.zeros((HIDDEN,), f32), jnp.zeros((HIDDEN,), f32),
        jnp.zeros((HIDDEN,), f32)], axis=0)
    h01 = jnp.pad(h[0:2], ((0, 6), (0, 0)))

    dstH = pl.BlockSpec((TI, HIDDEN), lambda t, s0, s1: (t, 0))
    dst8 = pl.BlockSpec((TI, 8), lambda t, s0, s1: (t, 0))
    full = lambda r, c: pl.BlockSpec((r, c), lambda t, s0, s1: (0, 0))

    grid_spec = pltpu.PrefetchScalarGridSpec(
        num_scalar_prefetch=2,
        grid=(num_t,),
        in_specs=[dstH, dstH, dstH, dstH, dstH, dst8, dst8,
                  full(npad, HIDDEN), full(npad, HIDDEN), full(npad, HIDDEN),
                  full(npad, 8), full(npad, 8),
                  full(HIDDEN, 3 * HIDDEN), full(HIDDEN, HIDDEN),
                  full(HIDDEN, HIDDEN), full(HIDDEN, HIDDEN),
                  full(HIDDEN, HIDDEN), full(HIDDEN, HIDDEN),
                  full(HIDDEN, HIDDEN), full(16, HIDDEN), full(8, HIDDEN)],
        out_specs=[dstH, dst8],
    )
    hout, xout = pl.pallas_call(
        _pair_kernel,
        grid_spec=grid_spec,
        out_shape=(jax.ShapeDtypeStruct((npad, HIDDEN), f32),
                   jax.ShapeDtypeStruct((npad, 8), f32)),
        compiler_params=pltpu.CompilerParams(
            dimension_semantics=("arbitrary",),
            vmem_limit_bytes=128 * 1024 * 1024),
    )(start_t, num_s,
      q, bhk, bhv, bxv, hp, xp, auxd,
      ahk, ahv, axv, xp, auxd,
      w1r_rep, p['hk_W2'], p['hv_W2'], w2xv_rep,
      p['no_W1'][:HIDDEN], p['no_W1'][HIDDEN:], p['no_W2'], vec_pair, h01)

    return jnp.take(hout[:n], inv, axis=0), jnp.take(xout[:n, :3], inv, axis=0)


# speedup vs baseline: 1.1734x; 1.0183x over previous
import functools
import math

import jax
import jax.numpy as jnp
import numpy as np
from jax.experimental import pallas as pl
from jax.experimental.pallas import tpu as pltpu

HIDDEN = 128
NHEADS = 16
DH = HIDDEN // NHEADS
NRG = 16
TI = 32
TJ = 32

_OFF = np.linspace(0.0, 10.0, NRG).astype(np.float32)
_COEFF = np.float32(-0.5 / (_OFF[1] - _OFF[0]) ** 2)


def _ln(z, g, be):
    mu = z.mean(-1, keepdims=True)
    var = jnp.mean(z * z, axis=-1, keepdims=True) - mu * mu
    return (z - mu) * (1.0 / jnp.sqrt(var + 1e-5)) * g + be


def _pick_tile(n, cap=1024):
    best = 8
    for d in range(8, cap + 1, 8):
        if n % d == 0:
            best = d
    return best


def _pre_kernel(h_ref, w_ref, vec_ref, q_ref, ahk_ref, ahv_ref, axv_ref,
                bhk_ref, bhv_ref, bxv_ref):
    h = h_ref[...]
    w = w_ref
    b1q = vec_ref[0:1, :]
    gq = vec_ref[1:2, :]
    beq = vec_ref[2:3, :]
    b2q = vec_ref[3:4, :]
    dot = functools.partial(jnp.dot, preferred_element_type=jnp.float32)
    z = dot(h, w[0:128, :]) + b1q
    z = jax.nn.relu(_ln(z, gq, beq))
    q_ref[...] = (dot(z, w[128:256, :]) + b2q) * (1.0 / np.sqrt(DH))
    ahk_ref[...] = dot(h, w[256:384, :])
    ahv_ref[...] = dot(h, w[384:512, :])
    axv_ref[...] = dot(h, w[512:640, :])
    bhk_ref[...] = dot(h, w[640:768, :]) + vec_ref[4:5, :]
    bhv_ref[...] = dot(h, w[768:896, :]) + vec_ref[5:6, :]
    bxv_ref[...] = dot(h, w[896:1024, :]) + vec_ref[6:7, :]


def _pair_kernel(start_ref, num_ref,
                 q_ref, bhk_ref, bhv_ref, bxv_ref, hdst_ref, xdst_ref, auxd_ref,
                 ahk_ref, ahv_ref, axv_ref, xsrc_ref, auxs_ref,
                 w1r_ref, w2hk_ref, w2hv_ref, w2xv_ref,
                 wno1a_ref, wno1b_ref, wno2_ref, vec_ref, h01_ref,
                 hout_ref, xout_ref):
    t = pl.program_id(0)
    start = start_ref[t]
    nst = num_ref[t]

    dot = functools.partial(jnp.dot, preferred_element_type=jnp.float32)

    q = q_ref[...]
    bhk_i = bhk_ref[...]
    bhv_i = bhv_ref[...]
    bxv_i = bxv_ref[...]
    auxd = auxd_ref[...]
    b_i = auxd[:, 0:1]
    ml_i = auxd[:, 1:2]
    bi_eff = jnp.where(ml_i == 1, b_i, -1)
    bi_rep = jnp.broadcast_to(bi_eff, (TI, HIDDEN))
    xd = xdst_ref[...]
    xi0 = xd[:, 0:1]
    xi1 = xd[:, 1:2]
    xi2 = xd[:, 2:3]
    xi0_rep = jnp.broadcast_to(xi0, (TI, HIDDEN))
    xi1_rep = jnp.broadcast_to(xi1, (TI, HIDDEN))
    xi2_rep = jnp.broadcast_to(xi2, (TI, HIDDEN))

    g_hk = vec_ref[0:1, :]
    be_hk = vec_ref[1:2, :]
    b2_hk = vec_ref[2:3, :]
    g_hv = vec_ref[3:4, :]
    be_hv = vec_ref[4:5, :]
    b2_hv = vec_ref[5:6, :]
    g_xv = vec_ref[6:7, :]
    be_xv = vec_ref[7:8, :]
    b2_xv_rep = vec_ref[8:9, :]
    b1_no = vec_ref[9:10, :]
    g_no = vec_ref[10:11, :]
    be_no = vec_ref[11:12, :]
    b2_no = vec_ref[12:13, :]

    lane = jax.lax.broadcasted_iota(jnp.int32, (1, HIDDEN), 1)
    off_rep = (lane // DH).astype(jnp.float32) * np.float32(10.0 / (NRG - 1))
    rr = jax.lax.broadcasted_iota(jnp.int32, (HIDDEN, HIDDEN), 0) // DH
    cc = jax.lax.broadcasted_iota(jnp.int32, (HIDDEN, HIDDEN), 1) // DH
    s_head = (rr == cc).astype(jnp.float32)

    def body(s, carry):
        j0 = (start + s) * TJ
        auxj = auxs_ref[pl.ds(j0, TJ), :]
        bj_eff = jnp.where(auxj[:, 1:2] == 0, auxj[:, 0:1], -2)
        bj_rep = jnp.broadcast_to(bj_eff, (TJ, HIDDEN))
        mask3 = bi_rep[:, None, :] == bj_rep[None, :, :]
        return jax.lax.cond(
            jnp.any(mask3), lambda c: _tile(j0, mask3, c), lambda c: c, carry)

    def _tile(j0, mask3, carry):
        m, l, accv, ax0, ax1, ax2 = carry
        ahk = ahk_ref[pl.ds(j0, TJ), :]
        ahv = ahv_ref[pl.ds(j0, TJ), :]
        axv = axv_ref[pl.ds(j0, TJ), :]
        xj = xsrc_ref[pl.ds(j0, TJ), :]
        xj0_rep = jnp.broadcast_to(xj[:, 0:1], (TJ, HIDDEN))
        xj1_rep = jnp.broadcast_to(xj[:, 1:2], (TJ, HIDDEN))
        xj2_rep = jnp.broadcast_to(xj[:, 2:3], (TJ, HIDDEN))

        pen3 = jnp.where(mask3, 0.0, -jnp.inf)
        rel0 = xi0_rep[:, None, :] - xj0_rep[None, :, :]
        rel1 = xi1_rep[:, None, :] - xj1_rep[None, :, :]
        rel2 = xi2_rep[:, None, :] - xj2_rep[None, :, :]
        dist = jnp.sqrt(rel0 * rel0 + rel1 * rel1 + rel2 * rel2)
        rf = jnp.exp(_COEFF * (dist - off_rep[None, :, :]) ** 2)
        r_all = dot(rf.reshape(TI * TJ, HIDDEN), w1r_ref[...])
        r_all = r_all.reshape(TI, TJ, 3 * HIDDEN)

        z = r_all[:, :, 0:HIDDEN] + ahk[None, :, :] + bhk_i[:, None, :]
        z = jax.nn.relu(_ln(z, g_hk, be_hk))
        k = dot(z.reshape(TI * TJ, HIDDEN), w2hk_ref[...]) + b2_hk
        qk = (q[:, None, :] * k.reshape(TI, TJ, HIDDEN)).reshape(TI * TJ, HIDDEN)
        logit = dot(qk, s_head).reshape(TI, TJ, HIDDEN) + pen3

        tmax = logit.max(axis=1)
        m_new = jnp.maximum(m, tmax)
        m_sub = jnp.where(jnp.isfinite(m_new), m_new, 0.0)
        m_sub_prev = jnp.where(jnp.isfinite(m), m, 0.0)
        scale = jnp.where(jnp.isfinite(m), jnp.exp(m_sub_prev - m_sub), 0.0)
        p = jnp.exp(logit - m_sub[:, None, :])
        l_new = l * scale + p.sum(axis=1)

        z = r_all[:, :, HIDDEN:2 * HIDDEN] + ahv[None, :, :] + bhv_i[:, None, :]
        z = jax.nn.relu(_ln(z, g_hv, be_hv))
        v = dot(z.reshape(TI * TJ, HIDDEN), w2hv_ref[...]) + b2_hv
        pv = (p * v.reshape(TI, TJ, HIDDEN)).sum(axis=1)
        accv_new = accv * scale + pv

        z = r_all[:, :, 2 * HIDDEN:3 * HIDDEN] + axv[None, :, :] + bxv_i[:, None, :]
        z = jax.nn.relu(_ln(z, g_xv, be_xv))
        xv = dot(z.reshape(TI * TJ, HIDDEN), w2xv_ref[...]) + b2_xv_rep
        w = p * xv.reshape(TI, TJ, HIDDEN)
        ax0_new = ax0 * scale + (w * rel0).sum(axis=1)
        ax1_new = ax1 * scale + (w * rel1).sum(axis=1)
        ax2_new = ax2 * scale + (w * rel2).sum(axis=1)
        return m_new, l_new, accv_new, ax0_new, ax1_new, ax2_new

    init = (jnp.full((TI, HIDDEN), -jnp.inf, jnp.float32),
            jnp.zeros((TI, HIDDEN), jnp.float32),
            jnp.zeros((TI, HIDDEN), jnp.float32),
            jnp.zeros((TI, HIDDEN), jnp.float32),
            jnp.zeros((TI, HIDDEN), jnp.float32),
            jnp.zeros((TI, HIDDEN), jnp.float32))
    m, l, accv, ax0, ax1, ax2 = jax.lax.fori_loop(0, nst, body, init)

    denom = l + 1e-16
    attn_out = accv / denom
    dx0 = (ax0 / denom).sum(axis=1, keepdims=True) * np.float32(1.0 / HIDDEN)
    dx1 = (ax1 / denom).sum(axis=1, keepdims=True) * np.float32(1.0 / HIDDEN)
    dx2 = (ax2 / denom).sum(axis=1, keepdims=True) * np.float32(1.0 / HIDDEN)
    xout_ref[...] = jnp.concatenate(
        [xi0 + dx0, xi1 + dx1, xi2 + dx2, xd[:, 3:8]], axis=1)

    hml = jnp.where(ml_i == 1, h01_ref[1:2, :], h01_ref[0:1, :])
    z = (dot(attn_out, wno1a_ref[...]) + dot(hml, wno1b_ref[...]) + b1_no)
    z = jax.nn.relu(_ln(z, g_no, be_no))
    hout_ref[...] = dot(z, wno2_ref[...]) + b2_no + hdst_ref[...]


def kernel(h, x, params, batch, edge_index, mask_ligand):
    n = h.shape[0]
    npad = -(-n // TJ) * TJ
    num_t = npad // TI
    pad = npad - n

    f32 = jnp.float32
    h = h.astype(f32)
    x = x.astype(f32)
    b32 = batch.astype(jnp.int32)
    ml32 = mask_ligand.astype(jnp.int32)

    perm = jnp.argsort(b32 * 2 + ml32, stable=True)
    inv = jnp.zeros((n,), jnp.int32).at[perm].set(
        jnp.arange(n, dtype=jnp.int32))
    hs = jnp.take(h, perm, axis=0)
    xs = jnp.take(x, perm, axis=0)
    bs = jnp.take(b32, perm)
    mls = jnp.take(ml32, perm)

    sentinel = np.int32(1 << 20)
    hp = jnp.pad(hs, ((0, pad), (0, 0)))
    xp = jnp.pad(xs, ((0, pad), (0, 5)))
    batch_p = jnp.pad(bs, (0, pad), constant_values=sentinel)
    ml_p = jnp.pad(mls, (0, pad))
    zcol = jnp.zeros((npad, 6), jnp.int32)
    auxd = jnp.concatenate([batch_p[:, None], ml_p[:, None], zcol], axis=1)

    p = params
    wpack = jnp.concatenate([
        p['hq_W1'], p['hq_W2'],
        p['hk_W1'][NRG:NRG + HIDDEN], p['hv_W1'][NRG:NRG + HIDDEN],
        p['xv_W1'][NRG:NRG + HIDDEN],
        p['hk_W1'][NRG + HIDDEN:], p['hv_W1'][NRG + HIDDEN:],
        p['xv_W1'][NRG + HIDDEN:],
    ], axis=0)
    vec_pre = jnp.stack([
        p['hq_b1'], p['hq_g'], p['hq_be'], p['hq_b2'],
        p['hk_b1'], p['hv_b1'], p['xv_b1'], jnp.zeros((HIDDEN,), f32)], axis=0)

    tp = _pick_tile(npad)
    node_out = tuple(jax.ShapeDtypeStruct((npad, HIDDEN), f32) for _ in range(7))
    blk = pl.BlockSpec((tp, HIDDEN), lambda i: (i, 0))
    q, ahk, ahv, axv, bhk, bhv, bxv = pl.pallas_call(
        _pre_kernel,
        grid=(npad // tp,),
        in_specs=[blk,
                  pl.BlockSpec((8 * HIDDEN, HIDDEN), lambda i: (0, 0)),
                  pl.BlockSpec((8, HIDDEN), lambda i: (0, 0))],
        out_specs=tuple(blk for _ in range(7)),
        out_shape=node_out,
    )(hp, wpack, vec_pre)

    key_p = batch_p * 2 + ml_p
    i0 = jnp.arange(num_t, dtype=jnp.int32) * TI
    g_lo = batch_p[i0]
    g_hi = batch_p[i0 + TI - 1]
    row_lo = jnp.searchsorted(key_p, 2 * g_lo, side='left').astype(jnp.int32)
    row_hi = jnp.searchsorted(key_p, 2 * g_hi + 1, side='left').astype(jnp.int32)
    start_t = row_lo // TJ
    num_s = -(-row_hi // TJ) - start_t
    has_lig = ml_p.reshape(num_t, TI).max(axis=1) > 0
    num_s = jnp.where(has_lig, num_s, 0)

    w1r = jnp.concatenate(
        [p['hk_W1'][:NRG], p['hv_W1'][:NRG], p['xv_W1'][:NRG]], axis=1)
    w1r_rep = jnp.broadcast_to(
        w1r[:, None, :] * np.float32(1.0 / DH),
        (NRG, DH, 3 * HIDDEN)).reshape(HIDDEN, 3 * HIDDEN)
    w2xv_rep = jnp.broadcast_to(
        p['xv_W2'][:, :, None], (HIDDEN, NHEADS, DH)).reshape(HIDDEN, HIDDEN)
    b2xv_rep = jnp.broadcast_to(
        p['xv_b2'][:, None], (NHEADS, DH)).reshape(HIDDEN)
    vec_pair = jnp.stack([
        p['hk_g'], p['hk_be'], p['hk_b2'],
        p['hv_g'], p['hv_be'], p['hv_b2'],
        p['xv_g'], p['xv_be'], b2xv_rep,
        p['no_b1'], p['no_g'], p['no_be'], p['no_b2'],
        jnp.zeros((HIDDEN,), f32), jnp.zeros((HIDDEN,), f32),
        jnp.zeros((HIDDEN,), f32)], axis=0)
    h01 = jnp.pad(h[0:2], ((0, 6), (0, 0)))

    dstH = pl.BlockSpec((TI, HIDDEN), lambda t, s0, s1: (t, 0))
    dst8 = pl.BlockSpec((TI, 8), lambda t, s0, s1: (t, 0))
    full = lambda r, c: pl.BlockSpec((r, c), lambda t, s0, s1: (0, 0))

    grid_spec = pltpu.PrefetchScalarGridSpec(
        num_scalar_prefetch=2,
        grid=(num_t,),
        in_specs=[dstH, dstH, dstH, dstH, dstH, dst8, dst8,
                  full(npad, HIDDEN), full(npad, HIDDEN), full(npad, HIDDEN),
                  full(npad, 8), full(npad, 8),
                  full(HIDDEN, 3 * HIDDEN), full(HIDDEN, HIDDEN),
                  full(HIDDEN, HIDDEN), full(HIDDEN, HIDDEN),
                  full(HIDDEN, HIDDEN), full(HIDDEN, HIDDEN),
                  full(HIDDEN, HIDDEN), full(16, HIDDEN), full(8, HIDDEN)],
        out_specs=[dstH, dst8],
    )
    hout, xout = pl.pallas_call(
        _pair_kernel,
        grid_spec=grid_spec,
        out_shape=(jax.ShapeDtypeStruct((npad, HIDDEN), f32),
                   jax.ShapeDtypeStruct((npad, 8), f32)),
        compiler_params=pltpu.CompilerParams(
            dimension_semantics=("arbitrary",),
            vmem_limit_bytes=128 * 1024 * 1024),
    )(start_t, num_s,
      q, bhk, bhv, bxv, hp, xp, auxd,
      ahk, ahv, axv, xp, auxd,
      w1r_rep, p['hk_W2'], p['hv_W2'], w2xv_rep,
      p['no_W1'][:HIDDEN], p['no_W1'][HIDDEN:], p['no_W2'], vec_pair, h01)

    return jnp.take(hout[:n], inv, axis=0), jnp.take(xout[:n, :3], inv, axis=0)


# speedup vs baseline: 1.1906x; 1.0146x over previous
import functools
import math

import jax
import jax.numpy as jnp
import numpy as np
from jax.experimental import pallas as pl
from jax.experimental.pallas import tpu as pltpu

HIDDEN = 128
NHEADS = 16
DH = HIDDEN // NHEADS
NRG = 16
TI = 32
TJ = 32

_OFF = np.linspace(0.0, 10.0, NRG).astype(np.float32)
_COEFF = np.float32(-0.5 / (_OFF[1] - _OFF[0]) ** 2)


def _ln(z, g, be):
    mu = z.mean(-1, keepdims=True)
    var = jnp.mean(z * z, axis=-1, keepdims=True) - mu * mu
    return (z - mu) * (1.0 / jnp.sqrt(var + 1e-5)) * g + be


def _pick_tile(n, cap=1024):
    best = 8
    for d in range(8, cap + 1, 8):
        if n % d == 0:
            best = d
    return best


def _pre_kernel(h_ref, w_ref, vec_ref, q_ref, ahk_ref, ahv_ref, axv_ref,
                bhk_ref, bhv_ref, bxv_ref):
    h = h_ref[...]
    w = w_ref
    b1q = vec_ref[0:1, :]
    gq = vec_ref[1:2, :]
    beq = vec_ref[2:3, :]
    b2q = vec_ref[3:4, :]
    dot = functools.partial(jnp.dot, preferred_element_type=jnp.float32)
    z = dot(h, w[0:128, :]) + b1q
    z = jax.nn.relu(_ln(z, gq, beq))
    q_ref[...] = (dot(z, w[128:256, :]) + b2q) * (1.0 / np.sqrt(DH))
    ahk_ref[...] = dot(h, w[256:384, :])
    ahv_ref[...] = dot(h, w[384:512, :])
    axv_ref[...] = dot(h, w[512:640, :])
    bhk_ref[...] = dot(h, w[640:768, :]) + vec_ref[4:5, :]
    bhv_ref[...] = dot(h, w[768:896, :]) + vec_ref[5:6, :]
    bxv_ref[...] = dot(h, w[896:1024, :]) + vec_ref[6:7, :]


def _pair_kernel(start_ref, num_ref,
                 q_ref, bhk_ref, bhv_ref, bxv_ref, hdst_ref, xdst_ref, auxd_ref,
                 ahk_ref, ahv_ref, axv_ref, xsrc_ref, auxs_ref,
                 w1r_ref, w2hk_ref, w2hv_ref, w2xv_ref,
                 wno1a_ref, wno1b_ref, wno2_ref, vec_ref, h01_ref,
                 hout_ref, xout_ref):
    t = pl.program_id(0)
    start = start_ref[t]
    nst = num_ref[t]

    dot = functools.partial(jnp.dot, preferred_element_type=jnp.float32)

    q = q_ref[...]
    bhk_i = bhk_ref[...]
    bhv_i = bhv_ref[...]
    bxv_i = bxv_ref[...]
    auxd = auxd_ref[...]
    b_i = auxd[:, 0:1]
    ml_i = auxd[:, 1:2]
    bi_eff = jnp.where(ml_i == 1, b_i, -1)
    bi_rep = jnp.broadcast_to(bi_eff, (TI, HIDDEN))
    xd = xdst_ref[...]
    xi0 = xd[:, 0:1]
    xi1 = xd[:, 1:2]
    xi2 = xd[:, 2:3]
    xi0_rep = jnp.broadcast_to(xi0, (TI, HIDDEN))
    xi1_rep = jnp.broadcast_to(xi1, (TI, HIDDEN))
    xi2_rep = jnp.broadcast_to(xi2, (TI, HIDDEN))

    g_hk = vec_ref[0:1, :]
    be_hk = vec_ref[1:2, :]
    b2_hk = vec_ref[2:3, :]
    g_hv = vec_ref[3:4, :]
    be_hv = vec_ref[4:5, :]
    b2_hv = vec_ref[5:6, :]
    g_xv = vec_ref[6:7, :]
    be_xv = vec_ref[7:8, :]
    b2_xv_rep = vec_ref[8:9, :]
    b1_no = vec_ref[9:10, :]
    g_no = vec_ref[10:11, :]
    be_no = vec_ref[11:12, :]
    b2_no = vec_ref[12:13, :]

    lane = jax.lax.broadcasted_iota(jnp.int32, (1, HIDDEN), 1)
    off_rep = (lane // DH).astype(jnp.float32) * np.float32(10.0 / (NRG - 1))
    rr = jax.lax.broadcasted_iota(jnp.int32, (HIDDEN, HIDDEN), 0) // DH
    cc = jax.lax.broadcasted_iota(jnp.int32, (HIDDEN, HIDDEN), 1) // DH
    s_head = (rr == cc).astype(jnp.float32)
    qb2 = dot(q * b2_hk, s_head)

    def body(s, carry):
        j0 = (start + s) * TJ
        auxj = auxs_ref[pl.ds(j0, TJ), :]
        bj_eff = jnp.where(auxj[:, 1:2] == 0, auxj[:, 0:1], -2)
        bj_rep = jnp.broadcast_to(bj_eff, (TJ, HIDDEN))
        mask3 = bi_rep[:, None, :] == bj_rep[None, :, :]
        return jax.lax.cond(
            jnp.any(mask3), lambda c: _tile(j0, mask3, c), lambda c: c, carry)

    def _tile(j0, mask3, carry):
        m, l, accv, ax0, ax1, ax2 = carry
        ahk = ahk_ref[pl.ds(j0, TJ), :]
        ahv = ahv_ref[pl.ds(j0, TJ), :]
        axv = axv_ref[pl.ds(j0, TJ), :]
        xj = xsrc_ref[pl.ds(j0, TJ), :]
        xj0_rep = jnp.broadcast_to(xj[:, 0:1], (TJ, HIDDEN))
        xj1_rep = jnp.broadcast_to(xj[:, 1:2], (TJ, HIDDEN))
        xj2_rep = jnp.broadcast_to(xj[:, 2:3], (TJ, HIDDEN))

        rel0 = xi0_rep[:, None, :] - xj0_rep[None, :, :]
        rel1 = xi1_rep[:, None, :] - xj1_rep[None, :, :]
        rel2 = xi2_rep[:, None, :] - xj2_rep[None, :, :]
        dist = jnp.sqrt(rel0 * rel0 + rel1 * rel1 + rel2 * rel2)
        rf = jnp.exp(_COEFF * (dist - off_rep[None, :, :]) ** 2)
        r_all = dot(rf.reshape(TI * TJ, HIDDEN), w1r_ref[...])
        r_all = r_all.reshape(TI, TJ, 3 * HIDDEN)

        z = r_all[:, :, 0:HIDDEN] + ahk[None, :, :] + bhk_i[:, None, :]
        z = jax.nn.relu(_ln(z, g_hk, be_hk))
        k = dot(z.reshape(TI * TJ, HIDDEN), w2hk_ref[...])
        qk = (q[:, None, :] * k.reshape(TI, TJ, HIDDEN)).reshape(TI * TJ, HIDDEN)
        logit = jnp.where(
            mask3, dot(qk, s_head).reshape(TI, TJ, HIDDEN), -jnp.inf)

        tmax = logit.max(axis=1) + qb2
        m_new = jnp.maximum(m, tmax)
        m_sub = jnp.where(jnp.isfinite(m_new), m_new, 0.0)
        m_sub_prev = jnp.where(jnp.isfinite(m), m, 0.0)
        scale = jnp.where(jnp.isfinite(m), jnp.exp(m_sub_prev - m_sub), 0.0)
        p = jnp.exp(logit - (m_sub - qb2)[:, None, :])
        l_new = l * scale + p.sum(axis=1)

        z = r_all[:, :, HIDDEN:2 * HIDDEN] + ahv[None, :, :] + bhv_i[:, None, :]
        z = jax.nn.relu(_ln(z, g_hv, be_hv))
        v = dot(z.reshape(TI * TJ, HIDDEN), w2hv_ref[...])
        pv = (p * v.reshape(TI, TJ, HIDDEN)).sum(axis=1)
        accv_new = accv * scale + pv

        z = r_all[:, :, 2 * HIDDEN:3 * HIDDEN] + axv[None, :, :] + bxv_i[:, None, :]
        z = jax.nn.relu(_ln(z, g_xv, be_xv))
        xv = dot(z.reshape(TI * TJ, HIDDEN), w2xv_ref[...]) + b2_xv_rep
        w = p * xv.reshape(TI, TJ, HIDDEN)
        ax0_new = ax0 * scale + (w * rel0).sum(axis=1)
        ax1_new = ax1 * scale + (w * rel1).sum(axis=1)
        ax2_new = ax2 * scale + (w * rel2).sum(axis=1)
        return m_new, l_new, accv_new, ax0_new, ax1_new, ax2_new

    init = (jnp.full((TI, HIDDEN), -jnp.inf, jnp.float32),
            jnp.zeros((TI, HIDDEN), jnp.float32),
            jnp.zeros((TI, HIDDEN), jnp.float32),
            jnp.zeros((TI, HIDDEN), jnp.float32),
            jnp.zeros((TI, HIDDEN), jnp.float32),
            jnp.zeros((TI, HIDDEN), jnp.float32))
    m, l, accv, ax0, ax1, ax2 = jax.lax.fori_loop(0, nst, body, init)

    denom = l + 1e-16
    attn_out = accv / denom + b2_hv * (l / denom)
    dx0 = (ax0 / denom).sum(axis=1, keepdims=True) * np.float32(1.0 / HIDDEN)
    dx1 = (ax1 / denom).sum(axis=1, keepdims=True) * np.float32(1.0 / HIDDEN)
    dx2 = (ax2 / denom).sum(axis=1, keepdims=True) * np.float32(1.0 / HIDDEN)
    xout_ref[...] = jnp.concatenate(
        [xi0 + dx0, xi1 + dx1, xi2 + dx2, xd[:, 3:8]], axis=1)

    hml = jnp.where(ml_i == 1, h01_ref[1:2, :], h01_ref[0:1, :])
    z = (dot(attn_out, wno1a_ref[...]) + dot(hml, wno1b_ref[...]) + b1_no)
    z = jax.nn.relu(_ln(z, g_no, be_no))
    hout_ref[...] = dot(z, wno2_ref[...]) + b2_no + hdst_ref[...]


def kernel(h, x, params, batch, edge_index, mask_ligand):
    n = h.shape[0]
    npad = -(-n // TJ) * TJ
    num_t = npad // TI
    pad = npad - n

    f32 = jnp.float32
    h = h.astype(f32)
    x = x.astype(f32)
    b32 = batch.astype(jnp.int32)
    ml32 = mask_ligand.astype(jnp.int32)

    perm = jnp.argsort(b32 * 2 + ml32, stable=True)
    inv = jnp.zeros((n,), jnp.int32).at[perm].set(
        jnp.arange(n, dtype=jnp.int32))
    hs = jnp.take(h, perm, axis=0)
    xs = jnp.take(x, perm, axis=0)
    bs = jnp.take(b32, perm)
    mls = jnp.take(ml32, perm)

    sentinel = np.int32(1 << 20)
    hp = jnp.pad(hs, ((0, pad), (0, 0)))
    xp = jnp.pad(xs, ((0, pad), (0, 5)))
    batch_p = jnp.pad(bs, (0, pad), constant_values=sentinel)
    ml_p = jnp.pad(mls, (0, pad))
    zcol = jnp.zeros((npad, 6), jnp.int32)
    auxd = jnp.concatenate([batch_p[:, None], ml_p[:, None], zcol], axis=1)

    p = params
    wpack = jnp.concatenate([
        p['hq_W1'], p['hq_W2'],
        p['hk_W1'][NRG:NRG + HIDDEN], p['hv_W1'][NRG:NRG + HIDDEN],
        p['xv_W1'][NRG:NRG + HIDDEN],
        p['hk_W1'][NRG + HIDDEN:], p['hv_W1'][NRG + HIDDEN:],
        p['xv_W1'][NRG + HIDDEN:],
    ], axis=0)
    vec_pre = jnp.stack([
        p['hq_b1'], p['hq_g'], p['hq_be'], p['hq_b2'],
        p['hk_b1'], p['hv_b1'], p['xv_b1'], jnp.zeros((HIDDEN,), f32)], axis=0)

    tp = _pick_tile(npad)
    node_out = tuple(jax.ShapeDtypeStruct((npad, HIDDEN), f32) for _ in range(7))
    blk = pl.BlockSpec((tp, HIDDEN), lambda i: (i, 0))
    q, ahk, ahv, axv, bhk, bhv, bxv = pl.pallas_call(
        _pre_kernel,
        grid=(npad // tp,),
        in_specs=[blk,
                  pl.BlockSpec((8 * HIDDEN, HIDDEN), lambda i: (0, 0)),
                  pl.BlockSpec((8, HIDDEN), lambda i: (0, 0))],
        out_specs=tuple(blk for _ in range(7)),
        out_shape=node_out,
    )(hp, wpack, vec_pre)

    key_p = batch_p * 2 + ml_p
    i0 = jnp.arange(num_t, dtype=jnp.int32) * TI
    g_lo = batch_p[i0]
    g_hi = batch_p[i0 + TI - 1]
    row_lo = jnp.searchsorted(key_p, 2 * g_lo, side='left').astype(jnp.int32)
    row_hi = jnp.searchsorted(key_p, 2 * g_hi + 1, side='left').astype(jnp.int32)
    start_t = row_lo // TJ
    num_s = -(-row_hi // TJ) - start_t
    has_lig = ml_p.reshape(num_t, TI).max(axis=1) > 0
    num_s = jnp.where(has_lig, num_s, 0)

    w1r = jnp.concatenate(
        [p['hk_W1'][:NRG], p['hv_W1'][:NRG], p['xv_W1'][:NRG]], axis=1)
    w1r_rep = jnp.broadcast_to(
        w1r[:, None, :] * np.float32(1.0 / DH),
        (NRG, DH, 3 * HIDDEN)).reshape(HIDDEN, 3 * HIDDEN)
    w2xv_rep = jnp.broadcast_to(
        p['xv_W2'][:, :, None], (HIDDEN, NHEADS, DH)).reshape(HIDDEN, HIDDEN)
    b2xv_rep = jnp.broadcast_to(
        p['xv_b2'][:, None], (NHEADS, DH)).reshape(HIDDEN)
    vec_pair = jnp.stack([
        p['hk_g'], p['hk_be'], p['hk_b2'],
        p['hv_g'], p['hv_be'], p['hv_b2'],
        p['xv_g'], p['xv_be'], b2xv_rep,
        p['no_b1'], p['no_g'], p['no_be'], p['no_b2'],
        jnp.zeros((HIDDEN,), f32), jnp.zeros((HIDDEN,), f32),
        jnp.zeros((HIDDEN,), f32)], axis=0)
    h01 = jnp.pad(h[0:2], ((0, 6), (0, 0)))

    dstH = pl.BlockSpec((TI, HIDDEN), lambda t, s0, s1: (t, 0))
    dst8 = pl.BlockSpec((TI, 8), lambda t, s0, s1: (t, 0))
    full = lambda r, c: pl.BlockSpec((r, c), lambda t, s0, s1: (0, 0))

    grid_spec = pltpu.PrefetchScalarGridSpec(
        num_scalar_prefetch=2,
        grid=(num_t,),
        in_specs=[dstH, dstH, dstH, dstH, dstH, dst8, dst8,
                  full(npad, HIDDEN), full(npad, HIDDEN), full(npad, HIDDEN),
                  full(npad, 8), full(npad, 8),
                  full(HIDDEN, 3 * HIDDEN), full(HIDDEN, HIDDEN),
                  full(HIDDEN, HIDDEN), full(HIDDEN, HIDDEN),
                  full(HIDDEN, HIDDEN), full(HIDDEN, HIDDEN),
                  full(HIDDEN, HIDDEN), full(16, HIDDEN), full(8, HIDDEN)],
        out_specs=[dstH, dst8],
    )
    hout, xout = pl.pallas_call(
        _pair_kernel,
        grid_spec=grid_spec,
        out_shape=(jax.ShapeDtypeStruct((npad, HIDDEN), f32),
                   jax.ShapeDtypeStruct((npad, 8), f32)),
        compiler_params=pltpu.CompilerParams(
            dimension_semantics=("arbitrary",),
            vmem_limit_bytes=128 * 1024 * 1024),
    )(start_t, num_s,
      q, bhk, bhv, bxv, hp, xp, auxd,
      ahk, ahv, axv, xp, auxd,
      w1r_rep, p['hk_W2'], p['hv_W2'], w2xv_rep,
      p['no_W1'][:HIDDEN], p['no_W1'][HIDDEN:], p['no_W2'], vec_pair, h01)

    return jnp.take(hout[:n], inv, axis=0), jnp.take(xout[:n, :3], inv, axis=0)


# speedup vs baseline: 1.4229x; 1.1951x over previous
import functools
import math

import jax
import jax.numpy as jnp
import numpy as np
from jax.experimental import pallas as pl
from jax.experimental.pallas import tpu as pltpu

HIDDEN = 128
NHEADS = 16
DH = HIDDEN // NHEADS
NRG = 16
TI = 32
TJ = 32

_OFF = np.linspace(0.0, 10.0, NRG).astype(np.float32)
_COEFF = np.float32(-0.5 / (_OFF[1] - _OFF[0]) ** 2)


def _ln(z, g, be):
    mu = z.mean(-1, keepdims=True)
    var = jnp.mean(z * z, axis=-1, keepdims=True) - mu * mu
    return (z - mu) * (1.0 / jnp.sqrt(var + 1e-5)) * g + be


def _pick_tile(n, cap=1024):
    best = 8
    for d in range(8, cap + 1, 8):
        if n % d == 0:
            best = d
    return best


def _pre_kernel(h_ref, w_ref, vec_ref, q_ref, ahk_ref, ahv_ref, axv_ref,
                bhk_ref, bhv_ref, bxv_ref):
    h = h_ref[...]
    w = w_ref
    b1q = vec_ref[0:1, :]
    gq = vec_ref[1:2, :]
    beq = vec_ref[2:3, :]
    b2q = vec_ref[3:4, :]
    dot = functools.partial(jnp.dot, preferred_element_type=jnp.float32)
    z = dot(h, w[0:128, :]) + b1q
    z = jax.nn.relu(_ln(z, gq, beq))
    q_ref[...] = (dot(z, w[128:256, :]) + b2q) * (1.0 / np.sqrt(DH))
    ahk_ref[...] = dot(h, w[256:384, :])
    ahv_ref[...] = dot(h, w[384:512, :])
    axv_ref[...] = dot(h, w[512:640, :])
    bhk_ref[...] = dot(h, w[640:768, :]) + vec_ref[4:5, :]
    bhv_ref[...] = dot(h, w[768:896, :]) + vec_ref[5:6, :]
    bxv_ref[...] = dot(h, w[896:1024, :]) + vec_ref[6:7, :]


def _pair_kernel(s1_ref, n1_ref, s2_ref, n2_ref,
                 q_ref, bhk_ref, bhv_ref, bxv_ref, hdst_ref, xdst_ref, auxd_ref,
                 ahk_ref, ahv_ref, axv_ref, xsrc_ref, auxs_ref,
                 w1r_ref, w2hk_ref, w2hv_ref, w2xv_ref,
                 wno1a_ref, wno1b_ref, wno2_ref, vec_ref, h01_ref,
                 hout_ref, xout_ref):
    t = pl.program_id(0)

    dot = functools.partial(jnp.dot, preferred_element_type=jnp.float32)

    q = q_ref[...]
    bhk_i = bhk_ref[...]
    bhv_i = bhv_ref[...]
    bxv_i = bxv_ref[...]
    auxd = auxd_ref[...]
    b_i = auxd[:, 0:1]
    ml_i = auxd[:, 1:2]
    bi_eff = jnp.where(ml_i == 1, b_i, -1)
    bi_rep = jnp.broadcast_to(bi_eff, (TI, HIDDEN))
    xd = xdst_ref[...]
    xi0 = xd[:, 0:1]
    xi1 = xd[:, 1:2]
    xi2 = xd[:, 2:3]
    xi0_rep = jnp.broadcast_to(xi0, (TI, HIDDEN))
    xi1_rep = jnp.broadcast_to(xi1, (TI, HIDDEN))
    xi2_rep = jnp.broadcast_to(xi2, (TI, HIDDEN))

    g_hk = vec_ref[0:1, :]
    be_hk = vec_ref[1:2, :]
    b2_hk = vec_ref[2:3, :]
    g_hv = vec_ref[3:4, :]
    be_hv = vec_ref[4:5, :]
    b2_hv = vec_ref[5:6, :]
    g_xv = vec_ref[6:7, :]
    be_xv = vec_ref[7:8, :]
    b2_xv_rep = vec_ref[8:9, :]
    b1_no = vec_ref[9:10, :]
    g_no = vec_ref[10:11, :]
    be_no = vec_ref[11:12, :]
    b2_no = vec_ref[12:13, :]

    lane = jax.lax.broadcasted_iota(jnp.int32, (1, HIDDEN), 1)
    off_rep = (lane // DH).astype(jnp.float32) * np.float32(10.0 / (NRG - 1))
    rr = jax.lax.broadcasted_iota(jnp.int32, (HIDDEN, HIDDEN), 0) // DH
    cc = jax.lax.broadcasted_iota(jnp.int32, (HIDDEN, HIDDEN), 1) // DH
    s_head = (rr == cc).astype(jnp.float32)
    qb2 = dot(q * b2_hk, s_head)

    def _tile(j0, carry):
        m, l, accv, ax0, ax1, ax2 = carry
        auxj = auxs_ref[pl.ds(j0, TJ), :]
        bj_eff = jnp.where(auxj[:, 1:2] == 0, auxj[:, 0:1], -2)
        bj_rep = jnp.broadcast_to(bj_eff, (TJ, HIDDEN))
        mask3 = bi_rep[:, None, :] == bj_rep[None, :, :]
        ahk = ahk_ref[pl.ds(j0, TJ), :]
        ahv = ahv_ref[pl.ds(j0, TJ), :]
        axv = axv_ref[pl.ds(j0, TJ), :]
        xj = xsrc_ref[pl.ds(j0, TJ), :]
        xj0_rep = jnp.broadcast_to(xj[:, 0:1], (TJ, HIDDEN))
        xj1_rep = jnp.broadcast_to(xj[:, 1:2], (TJ, HIDDEN))
        xj2_rep = jnp.broadcast_to(xj[:, 2:3], (TJ, HIDDEN))

        rel0 = xi0_rep[:, None, :] - xj0_rep[None, :, :]
        rel1 = xi1_rep[:, None, :] - xj1_rep[None, :, :]
        rel2 = xi2_rep[:, None, :] - xj2_rep[None, :, :]
        dist = jnp.sqrt(rel0 * rel0 + rel1 * rel1 + rel2 * rel2)
        rf = jnp.exp(_COEFF * (dist - off_rep[None, :, :]) ** 2)
        r_all = dot(rf.reshape(TI * TJ, HIDDEN), w1r_ref[...])
        r_all = r_all.reshape(TI, TJ, 3 * HIDDEN)

        z = r_all[:, :, 0:HIDDEN] + ahk[None, :, :] + bhk_i[:, None, :]
        z = jax.nn.relu(_ln(z, g_hk, be_hk))
        k = dot(z.reshape(TI * TJ, HIDDEN), w2hk_ref[...])
        qk = (q[:, None, :] * k.reshape(TI, TJ, HIDDEN)).reshape(TI * TJ, HIDDEN)
        logit = jnp.where(
            mask3, dot(qk, s_head).reshape(TI, TJ, HIDDEN), -jnp.inf)

        tmax = logit.max(axis=1) + qb2
        m_new = jnp.maximum(m, tmax)
        m_sub = jnp.where(jnp.isfinite(m_new), m_new, 0.0)
        m_sub_prev = jnp.where(jnp.isfinite(m), m, 0.0)
        scale = jnp.where(jnp.isfinite(m), jnp.exp(m_sub_prev - m_sub), 0.0)
        p = jnp.exp(logit - (m_sub - qb2)[:, None, :])
        l_new = l * scale + p.sum(axis=1)

        z = r_all[:, :, HIDDEN:2 * HIDDEN] + ahv[None, :, :] + bhv_i[:, None, :]
        z = jax.nn.relu(_ln(z, g_hv, be_hv))
        v = dot(z.reshape(TI * TJ, HIDDEN), w2hv_ref[...])
        pv = (p * v.reshape(TI, TJ, HIDDEN)).sum(axis=1)
        accv_new = accv * scale + pv

        z = r_all[:, :, 2 * HIDDEN:3 * HIDDEN] + axv[None, :, :] + bxv_i[:, None, :]
        z = jax.nn.relu(_ln(z, g_xv, be_xv))
        xv = dot(z.reshape(TI * TJ, HIDDEN), w2xv_ref[...]) + b2_xv_rep
        w = p * xv.reshape(TI, TJ, HIDDEN)
        ax0_new = ax0 * scale + (w * rel0).sum(axis=1)
        ax1_new = ax1 * scale + (w * rel1).sum(axis=1)
        ax2_new = ax2 * scale + (w * rel2).sum(axis=1)
        return m_new, l_new, accv_new, ax0_new, ax1_new, ax2_new

    init = (jnp.full((TI, HIDDEN), -jnp.inf, jnp.float32),
            jnp.zeros((TI, HIDDEN), jnp.float32),
            jnp.zeros((TI, HIDDEN), jnp.float32),
            jnp.zeros((TI, HIDDEN), jnp.float32),
            jnp.zeros((TI, HIDDEN), jnp.float32),
            jnp.zeros((TI, HIDDEN), jnp.float32))
    s1 = s1_ref[t]
    s2 = s2_ref[t]
    carry = jax.lax.fori_loop(
        0, n1_ref[t], lambda s, c: _tile((s1 + s) * TJ, c), init)
    m, l, accv, ax0, ax1, ax2 = jax.lax.fori_loop(
        0, n2_ref[t], lambda s, c: _tile((s2 + s) * TJ, c), carry)

    denom = l + 1e-16
    attn_out = accv / denom + b2_hv * (l / denom)
    dx0 = (ax0 / denom).sum(axis=1, keepdims=True) * np.float32(1.0 / HIDDEN)
    dx1 = (ax1 / denom).sum(axis=1, keepdims=True) * np.float32(1.0 / HIDDEN)
    dx2 = (ax2 / denom).sum(axis=1, keepdims=True) * np.float32(1.0 / HIDDEN)
    xout_ref[...] = jnp.concatenate(
        [xi0 + dx0, xi1 + dx1, xi2 + dx2, xd[:, 3:8]], axis=1)

    hml = jnp.where(ml_i == 1, h01_ref[1:2, :], h01_ref[0:1, :])
    z = (dot(attn_out, wno1a_ref[...]) + dot(hml, wno1b_ref[...]) + b1_no)
    z = jax.nn.relu(_ln(z, g_no, be_no))
    hout_ref[...] = dot(z, wno2_ref[...]) + b2_no + hdst_ref[...]


def kernel(h, x, params, batch, edge_index, mask_ligand):
    n = h.shape[0]
    npad = -(-n // TJ) * TJ
    num_t = npad // TI
    pad = npad - n

    f32 = jnp.float32
    h = h.astype(f32)
    x = x.astype(f32)
    b32 = batch.astype(jnp.int32)
    ml32 = mask_ligand.astype(jnp.int32)

    perm = jnp.argsort(b32 * 2 + ml32, stable=True)
    inv = jnp.zeros((n,), jnp.int32).at[perm].set(
        jnp.arange(n, dtype=jnp.int32))
    hs = jnp.take(h, perm, axis=0)
    xs = jnp.take(x, perm, axis=0)
    bs = jnp.take(b32, perm)
    mls = jnp.take(ml32, perm)

    sentinel = np.int32(1 << 20)
    hp = jnp.pad(hs, ((0, pad), (0, 0)))
    xp = jnp.pad(xs, ((0, pad), (0, 5)))
    batch_p = jnp.pad(bs, (0, pad), constant_values=sentinel)
    ml_p = jnp.pad(mls, (0, pad))
    zcol = jnp.zeros((npad, 6), jnp.int32)
    auxd = jnp.concatenate([batch_p[:, None], ml_p[:, None], zcol], axis=1)

    p = params
    wpack = jnp.concatenate([
        p['hq_W1'], p['hq_W2'],
        p['hk_W1'][NRG:NRG + HIDDEN], p['hv_W1'][NRG:NRG + HIDDEN],
        p['xv_W1'][NRG:NRG + HIDDEN],
        p['hk_W1'][NRG + HIDDEN:], p['hv_W1'][NRG + HIDDEN:],
        p['xv_W1'][NRG + HIDDEN:],
    ], axis=0)
    vec_pre = jnp.stack([
        p['hq_b1'], p['hq_g'], p['hq_be'], p['hq_b2'],
        p['hk_b1'], p['hv_b1'], p['xv_b1'], jnp.zeros((HIDDEN,), f32)], axis=0)

    tp = _pick_tile(npad)
    node_out = tuple(jax.ShapeDtypeStruct((npad, HIDDEN), f32) for _ in range(7))
    blk = pl.BlockSpec((tp, HIDDEN), lambda i: (i, 0))
    q, ahk, ahv, axv, bhk, bhv, bxv = pl.pallas_call(
        _pre_kernel,
        grid=(npad // tp,),
        in_specs=[blk,
                  pl.BlockSpec((8 * HIDDEN, HIDDEN), lambda i: (0, 0)),
                  pl.BlockSpec((8, HIDDEN), lambda i: (0, 0))],
        out_specs=tuple(blk for _ in range(7)),
        out_shape=node_out,
    )(hp, wpack, vec_pre)

    key_p = batch_p * 2 + ml_p
    g_lo = jnp.min(jnp.where(ml_p == 1, batch_p, sentinel).reshape(num_t, TI),
                   axis=1)
    g_hi = jnp.max(jnp.where(ml_p == 1, batch_p, -1).reshape(num_t, TI),
                   axis=1)
    has_lig = g_hi >= 0
    r1_lo = jnp.searchsorted(key_p, 2 * g_lo, side='left').astype(jnp.int32)
    r1_hi = jnp.searchsorted(key_p, 2 * g_lo + 1, side='left').astype(jnp.int32)
    r2_lo = jnp.searchsorted(key_p, 2 * g_lo + 2, side='left').astype(jnp.int32)
    r2_hi = jnp.searchsorted(key_p, 2 * g_hi + 1, side='left').astype(jnp.int32)
    start1 = r1_lo // TJ
    num1 = jnp.where(has_lig & (r1_hi > r1_lo), -(-r1_hi // TJ) - start1, 0)
    start2 = r2_lo // TJ
    num2 = jnp.where(has_lig & (g_hi > g_lo) & (r2_hi > r2_lo),
                     -(-r2_hi // TJ) - start2, 0)

    w1r = jnp.concatenate(
        [p['hk_W1'][:NRG], p['hv_W1'][:NRG], p['xv_W1'][:NRG]], axis=1)
    w1r_rep = jnp.broadcast_to(
        w1r[:, None, :] * np.float32(1.0 / DH),
        (NRG, DH, 3 * HIDDEN)).reshape(HIDDEN, 3 * HIDDEN)
    w2xv_rep = jnp.broadcast_to(
        p['xv_W2'][:, :, None], (HIDDEN, NHEADS, DH)).reshape(HIDDEN, HIDDEN)
    b2xv_rep = jnp.broadcast_to(
        p['xv_b2'][:, None], (NHEADS, DH)).reshape(HIDDEN)
    vec_pair = jnp.stack([
        p['hk_g'], p['hk_be'], p['hk_b2'],
        p['hv_g'], p['hv_be'], p['hv_b2'],
        p['xv_g'], p['xv_be'], b2xv_rep,
        p['no_b1'], p['no_g'], p['no_be'], p['no_b2'],
        jnp.zeros((HIDDEN,), f32), jnp.zeros((HIDDEN,), f32),
        jnp.zeros((HIDDEN,), f32)], axis=0)
    h01 = jnp.pad(h[0:2], ((0, 6), (0, 0)))

    dstH = pl.BlockSpec((TI, HIDDEN), lambda t, *_: (t, 0))
    dst8 = pl.BlockSpec((TI, 8), lambda t, *_: (t, 0))
    full = lambda r, c: pl.BlockSpec((r, c), lambda t, *_: (0, 0))

    grid_spec = pltpu.PrefetchScalarGridSpec(
        num_scalar_prefetch=4,
        grid=(num_t,),
        in_specs=[dstH, dstH, dstH, dstH, dstH, dst8, dst8,
                  full(npad, HIDDEN), full(npad, HIDDEN), full(npad, HIDDEN),
                  full(npad, 8), full(npad, 8),
                  full(HIDDEN, 3 * HIDDEN), full(HIDDEN, HIDDEN),
                  full(HIDDEN, HIDDEN), full(HIDDEN, HIDDEN),
                  full(HIDDEN, HIDDEN), full(HIDDEN, HIDDEN),
                  full(HIDDEN, HIDDEN), full(16, HIDDEN), full(8, HIDDEN)],
        out_specs=[dstH, dst8],
    )
    hout, xout = pl.pallas_call(
        _pair_kernel,
        grid_spec=grid_spec,
        out_shape=(jax.ShapeDtypeStruct((npad, HIDDEN), f32),
                   jax.ShapeDtypeStruct((npad, 8), f32)),
        compiler_params=pltpu.CompilerParams(
            dimension_semantics=("arbitrary",),
            vmem_limit_bytes=128 * 1024 * 1024),
    )(start1, num1, start2, num2,
      q, bhk, bhv, bxv, hp, xp, auxd,
      ahk, ahv, axv, xp, auxd,
      w1r_rep, p['hk_W2'], p['hv_W2'], w2xv_rep,
      p['no_W1'][:HIDDEN], p['no_W1'][HIDDEN:], p['no_W2'], vec_pair, h01)

    return jnp.take(hout[:n], inv, axis=0), jnp.take(xout[:n, :3], inv, axis=0)
